# Initial kernel scaffold; baseline (speedup 1.0000x reference)
import functools

import jax
import jax.numpy as jnp
from jax import lax
from jax.experimental import pallas as pl
from jax.experimental.pallas import tpu as pltpu
from jax.experimental.pallas import tpu_sc as plsc

N_ROWS = 128
ROW = 32768
K = 6554
L = 16
K_PAD = 6560
NC, NS = 2, 16
NW = NC * NS
ROWS_PER_W = N_ROWS // NW

HIST = 2048
_RADIX_PASSES = ((0, 11), (11, 11), (22, 10))

_CNT_BASE = 1


def _key_from_bits(bits):
    flip = jnp.where(bits < 0, jnp.int32(0), jnp.int32(0x7FFFFFFF))
    return lax.bitwise_xor(bits, flip)


def _digit(k, shift, nbits):
    sh = jnp.full((L,), shift, jnp.int32)
    return lax.bitwise_and(
        lax.shift_right_logical(k, sh), jnp.full((L,), (1 << nbits) - 1, jnp.int32)
    )


def _body(x_hbm, out_hbm, buf_a, buf_b, buf_c, hist, outbuf):
    wid = lax.axis_index("s") * NC + lax.axis_index("c")

    def run_pass(load_fn, dst, shift, nbits):
        nbuck = 1 << nbits

        def zero_b(i, c):
            hist[pl.ds(i * L, L)] = jnp.zeros((L,), jnp.int32)
            return c

        lax.fori_loop(0, nbuck // L, zero_b, 0)

        def hist_b(i, c):
            d = _digit(load_fn(i), shift, nbits)
            cnt, last = plsc.scan_count(d)
            plsc.addupdate_scatter(hist, [d], cnt + (1 - _CNT_BASE), mask=last)
            return c

        lax.fori_loop(0, ROW // L, hist_b, 0)

        def scan_b(i, carry):
            h = hist[pl.ds(i * L, L)]
            inc = plsc.cumsum(h)
            hist[pl.ds(i * L, L)] = inc - h + carry
            return carry + jnp.sum(h)

        lax.fori_loop(0, nbuck // L, scan_b, jnp.int32(0))

        def perm_b(i, c):
            k = load_fn(i)
            d = _digit(k, shift, nbits)
            base = plsc.load_gather(hist, [d])
            cnt, last = plsc.scan_count(d)
            plsc.store_scatter(dst, [base + (cnt - _CNT_BASE)], k)
            plsc.addupdate_scatter(hist, [d], cnt + (1 - _CNT_BASE), mask=last)
            return c

        lax.fori_loop(0, ROW // L, perm_b, 0)

    def load_a(i):
        return _key_from_bits(plsc.bitcast(buf_a[pl.ds(i * L, L)], jnp.int32))

    def load_b(i):
        return buf_b[pl.ds(i * L, L)]

    def load_c(i):
        return buf_c[pl.ds(i * L, L)]

    def do_row(r, c):
        row = wid * ROWS_PER_W + r
        pltpu.sync_copy(x_hbm.at[row], buf_a)
        run_pass(load_a, buf_b, *_RADIX_PASSES[0])
        run_pass(load_b, buf_c, *_RADIX_PASSES[1])
        run_pass(load_c, buf_b, *_RADIX_PASSES[2])

        def out_b(i, cc):
            t = buf_b[pl.ds(i * L, L)]
            outbuf[pl.ds(i * L, L)] = plsc.bitcast(_key_from_bits(t), jnp.float32)
            return cc

        lax.fori_loop(0, K_PAD // L, out_b, 0)
        pltpu.sync_copy(outbuf.at[pl.ds(0, K)], out_hbm.at[row])
        return c

    lax.fori_loop(0, ROWS_PER_W, do_row, 0)


def kernel(inputs):
    mesh = plsc.VectorSubcoreMesh(
        core_axis_name="c", subcore_axis_name="s", num_cores=NC, num_subcores=NS
    )
    f = pl.kernel(
        _body,
        out_type=jax.ShapeDtypeStruct((N_ROWS, K), jnp.float32),
        mesh=mesh,
        scratch_types=[
            pltpu.VMEM((ROW,), jnp.float32),
            pltpu.VMEM((ROW,), jnp.int32),
            pltpu.VMEM((ROW,), jnp.int32),
            pltpu.VMEM((HIST,), jnp.int32),
            pltpu.VMEM((K_PAD,), jnp.float32),
        ],
    )
    return f(inputs)


# speedup vs baseline: 1.8567x; 1.8567x over previous
import functools

import jax
import jax.numpy as jnp
from jax import lax
from jax.experimental import pallas as pl
from jax.experimental.pallas import tpu as pltpu
from jax.experimental.pallas import tpu_sc as plsc

N_ROWS = 128
ROW = 32768
K = 6554
L = 16
K_PAD = 6656
NC, NS = 2, 16
NW = NC * NS
ROWS_PER_W = N_ROWS // NW

HIST = 2048
_RADIX_PASSES = ((0, 11), (11, 11), (22, 10))

_CNT_BASE = 1


def _key_from_bits(bits):
    flip = jnp.where(bits < 0, jnp.int32(0), jnp.int32(0x7FFFFFFF))
    return lax.bitwise_xor(bits, flip)


def _digit(k, shift, nbits):
    sh = jnp.full((L,), shift, jnp.int32)
    return lax.bitwise_and(
        lax.shift_right_logical(k, sh), jnp.full((L,), (1 << nbits) - 1, jnp.int32)
    )


def _body(x_hbm, out_hbm, buf_a, buf_b, buf_c, hist, outbuf):
    wid = lax.axis_index("s") * NC + lax.axis_index("c")

    def run_pass(load_fn, dst, shift, nbits):
        nbuck = 1 << nbits

        def zero_b(i, c):
            hist[pl.ds(i * L, L)] = jnp.zeros((L,), jnp.int32)
            return c

        lax.fori_loop(0, nbuck // L, zero_b, 0)

        def hist_b(i, c):
            d = _digit(load_fn(i), shift, nbits)
            cnt, last = plsc.scan_count(d)
            plsc.addupdate_scatter(hist, [d], cnt + (1 - _CNT_BASE), mask=last)
            return c

        lax.fori_loop(0, ROW // L, hist_b, 0)

        def scan_b(i, carry):
            h = hist[pl.ds(i * L, L)]
            inc = plsc.cumsum(h)
            hist[pl.ds(i * L, L)] = inc - h + carry
            return carry + jnp.sum(h)

        lax.fori_loop(0, nbuck // L, scan_b, jnp.int32(0))

        def perm_b(i, c):
            k = load_fn(i)
            d = _digit(k, shift, nbits)
            base = plsc.load_gather(hist, [d])
            cnt, last = plsc.scan_count(d)
            plsc.store_scatter(dst, [base + (cnt - _CNT_BASE)], k)
            plsc.addupdate_scatter(hist, [d], cnt + (1 - _CNT_BASE), mask=last)
            return c

        lax.fori_loop(0, ROW // L, perm_b, 0)

    def load_a(i):
        return _key_from_bits(buf_a[pl.ds(i * L, L)])

    def load_b(i):
        return buf_b[pl.ds(i * L, L)]

    def load_c(i):
        return buf_c[pl.ds(i * L, L)]

    def do_row(r, c):
        row = wid * ROWS_PER_W + r
        pltpu.sync_copy(x_hbm.at[row], buf_a)
        run_pass(load_a, buf_b, *_RADIX_PASSES[0])
        run_pass(load_b, buf_c, *_RADIX_PASSES[1])
        run_pass(load_c, buf_b, *_RADIX_PASSES[2])

        def out_b(i, cc):
            t = buf_b[pl.ds(i * L, L)]
            outbuf[pl.ds(i * L, L)] = _key_from_bits(t)
            return cc

        lax.fori_loop(0, K_PAD // L, out_b, 0)
        pltpu.sync_copy(outbuf, out_hbm.at[row])
        return c

    lax.fori_loop(0, ROWS_PER_W, do_row, 0)


def kernel(inputs):
    mesh = plsc.VectorSubcoreMesh(
        core_axis_name="c", subcore_axis_name="s", num_cores=NC, num_subcores=NS
    )
    f = pl.kernel(
        _body,
        out_type=jax.ShapeDtypeStruct((N_ROWS, K_PAD), jnp.int32),
        mesh=mesh,
        compiler_params=pltpu.CompilerParams(needs_layout_passes=False),
        scratch_types=[
            pltpu.VMEM((ROW,), jnp.int32),
            pltpu.VMEM((ROW,), jnp.int32),
            pltpu.VMEM((ROW,), jnp.int32),
            pltpu.VMEM((HIST,), jnp.int32),
            pltpu.VMEM((K_PAD,), jnp.int32),
        ],
    )
    bits = lax.bitcast_convert_type(inputs, jnp.int32)
    return lax.bitcast_convert_type(f(bits)[:, :K], jnp.float32)


# speedup vs baseline: 3.2338x; 1.7417x over previous
import functools

import jax
import jax.numpy as jnp
from jax import lax
from jax.experimental import pallas as pl
from jax.experimental.pallas import tpu as pltpu
from jax.experimental.pallas import tpu_sc as plsc

N_ROWS = 128
ROW = 32768
K = 6554
L = 16
K_PAD = 6656
NC, NS = 2, 16
NW = NC * NS
ROWS_PER_W = N_ROWS // NW

HIST = 2048

_CNT_BASE = 1


def _key_from_bits(bits):
    flip = jnp.where(bits < 0, jnp.int32(0), jnp.int32(0x7FFFFFFF))
    return lax.bitwise_xor(bits, flip)


def _digit(k, shift, nbits):
    sh = jnp.full((L,), shift, jnp.int32)
    return lax.bitwise_and(
        lax.shift_right_logical(k, sh), jnp.full((L,), (1 << nbits) - 1, jnp.int32)
    )


def _body(x_hbm, out_hbm, buf_a, buf_b, buf_c, hist, hist_d):
    wid = lax.axis_index("s") * NC + lax.axis_index("c")

    def zero_hist(nbuck):
        def zb(i, c):
            hist[pl.ds(i * L, L)] = jnp.zeros((L,), jnp.int32)
            return c

        lax.fori_loop(0, nbuck // L, zb, 0)

    def hist_sweep(load_fn, shift, nbits, n_iters):
        def hb(i, c):
            d = _digit(load_fn(i), shift, nbits)
            cnt, last = plsc.scan_count(d)
            plsc.addupdate_scatter(hist, [d], cnt, mask=last)
            return c

        lax.fori_loop(0, n_iters, hb, 0)

    def scan_hist(nbuck):
        def sb(i, carry):
            h = hist[pl.ds(i * L, L)]
            inc = plsc.cumsum(h)
            hist[pl.ds(i * L, L)] = inc - h + carry
            return carry + jnp.sum(h)

        lax.fori_loop(0, nbuck // L, sb, jnp.int32(0))

    def perm_sweep(load_fn, dst, shift, nbits, n_iters):
        def pb(i, c):
            k = load_fn(i)
            d = _digit(k, shift, nbits)
            base = plsc.load_gather(hist, [d])
            cnt, last = plsc.scan_count(d)
            plsc.store_scatter(dst, [base + (cnt - 1)], k)
            plsc.addupdate_scatter(hist, [d], cnt, mask=last)
            return c

        lax.fori_loop(0, n_iters, pb, 0)

    def load_a(i):
        return _key_from_bits(buf_a[pl.ds(i * L, L)])

    def load_b(i):
        return buf_b[pl.ds(i * L, L)]

    def load_c(i):
        return buf_c[pl.ds(i * L, L)]

    def do_row(r, c):
        row = wid * ROWS_PER_W + r
        pltpu.sync_copy(x_hbm.at[row], buf_a)

        zero_hist(HIST)
        hist_sweep(load_a, 21, 11, ROW // L)

        def scan_a(i, carry):
            c0, lcur = carry
            h = hist[pl.ds(i * L, L)]
            inc = plsc.cumsum(h)
            excl = inc - h + c0
            hist[pl.ds(i * L, L)] = excl
            hist_d[pl.ds(i * L, L)] = excl
            ends = inc + c0
            cand = jnp.where(ends >= K, ends, jnp.int32(1 << 30))
            return c0 + jnp.sum(h), jnp.minimum(lcur, jnp.min(cand))

        _, prefix_len = lax.fori_loop(
            0, HIST // L, scan_a, (jnp.int32(0), jnp.int32(1 << 30))
        )
        perm_sweep(load_a, buf_b, 21, 11, ROW // L)

        nv = (prefix_len + (L - 1)) // L

        zero_hist(HIST)
        hist_sweep(load_b, 0, 11, nv)
        scan_hist(HIST)
        perm_sweep(load_b, buf_c, 0, 11, nv)

        zero_hist(1024)
        hist_sweep(load_c, 11, 10, nv)
        scan_hist(1024)
        perm_sweep(load_c, buf_b, 11, 10, nv)

        def pd(i, cc):
            k = buf_b[pl.ds(i * L, L)]
            d = _digit(k, 21, 11)
            base = plsc.load_gather(hist_d, [d])
            cnt, last = plsc.scan_count(d)
            plsc.store_scatter(buf_c, [base + (cnt - 1)], _key_from_bits(k))
            plsc.addupdate_scatter(hist_d, [d], cnt, mask=last)
            return cc

        lax.fori_loop(0, nv, pd, 0)

        pltpu.sync_copy(buf_c.at[pl.ds(0, K_PAD)], out_hbm.at[row])
        return c

    lax.fori_loop(0, ROWS_PER_W, do_row, 0)


def kernel(inputs):
    mesh = plsc.VectorSubcoreMesh(
        core_axis_name="c", subcore_axis_name="s", num_cores=NC, num_subcores=NS
    )
    f = pl.kernel(
        _body,
        out_type=jax.ShapeDtypeStruct((N_ROWS, K_PAD), jnp.int32),
        mesh=mesh,
        compiler_params=pltpu.CompilerParams(needs_layout_passes=False),
        scratch_types=[
            pltpu.VMEM((ROW,), jnp.int32),
            pltpu.VMEM((ROW,), jnp.int32),
            pltpu.VMEM((ROW,), jnp.int32),
            pltpu.VMEM((HIST,), jnp.int32),
            pltpu.VMEM((HIST,), jnp.int32),
        ],
    )
    bits = lax.bitcast_convert_type(inputs, jnp.int32)
    return lax.bitcast_convert_type(f(bits)[:, :K], jnp.float32)
